# baseline (speedup 1.0000x reference)
import functools

import jax
import jax.numpy as jnp
from jax import lax
from jax.experimental import pallas as pl
from jax.experimental.pallas import tpu as pltpu
from jax.experimental.pallas import tpu_sc as plsc

_VOCAB = 100000
_D = 512
_NCL = 30
_B = 4096
_L = 200
_P = 32
_HALF = _L // 2

_info = plsc.get_sparse_core_info()
_NC, _NS = _info.num_cores, _info.num_subcores
_NW = _NC * _NS
_CPW = _B // _NW


def _proj_body(emb_ref, wp_ref, out_ref):
    out_ref[...] = jnp.dot(emb_ref[...], wp_ref[...],
                           preferred_element_type=jnp.float32
                           ).astype(jnp.bfloat16)


def _project(emb_table, wp):
    rows = 2000
    return pl.pallas_call(
        _proj_body,
        grid=(_VOCAB // rows,),
        in_specs=[
            pl.BlockSpec((rows, _D), lambda i: (i, 0)),
            pl.BlockSpec((_D, _P), lambda i: (0, 0)),
        ],
        out_specs=pl.BlockSpec((rows, _P), lambda i: (i, 0)),
        out_shape=jax.ShapeDtypeStruct((_VOCAB, _P), jnp.bfloat16),
    )(emb_table, wp)


_mesh = plsc.VectorSubcoreMesh(core_axis_name="c", subcore_axis_name="s")


@functools.partial(
    pl.kernel,
    out_type=jax.ShapeDtypeStruct((_B, _P), jnp.float32),
    mesh=_mesh,
    scratch_types=[
        pltpu.VMEM((_CPW, 2, _HALF), jnp.int32),
        pltpu.VMEM((2, _L, _P), jnp.bfloat16),
        pltpu.VMEM((_CPW, _P), jnp.float32),
        pltpu.SemaphoreType.DMA,
        pltpu.SemaphoreType.DMA,
    ],
    compiler_params=pltpu.CompilerParams(use_tc_tiling_on_sc=False,
                                         needs_layout_passes=False),
)
def _pool_kernel(tok_hbm, proj_hbm, out_hbm, idx_v, rows_v, acc_v, sem0, sem1):
    wid = lax.axis_index("s") * _NC + lax.axis_index("c")
    base = wid * _CPW
    pltpu.sync_copy(tok_hbm.at[pl.ds(base, _CPW)], idx_v)
    sems = (sem0, sem1)

    def fire(b, slot):
        pltpu.async_copy(proj_hbm.at[idx_v.at[b, 0]],
                         rows_v.at[slot, pl.ds(0, _HALF)], sems[slot])
        pltpu.async_copy(proj_hbm.at[idx_v.at[b, 1]],
                         rows_v.at[slot, pl.ds(_HALF, _HALF)], sems[slot])

    def wait_slot(slot):
        pltpu.make_async_copy(proj_hbm.at[pl.ds(0, _L)],
                              rows_v.at[slot], sems[slot]).wait()

    def accumulate(b, slot):
        def acc_body(i, carry):
            accs = list(carry)
            for j in range(8):
                r = i * 8 + j
                k = j % 4
                w = plsc.bitcast(rows_v[slot, r, 0:32], jnp.int32)
                lo = plsc.bitcast(lax.shift_left(w, 16), jnp.float32)
                hi = plsc.bitcast(jnp.bitwise_and(w, jnp.int32(-65536)),
                                  jnp.float32)
                accs[k] = accs[k] + lo
                accs[4 + k] = accs[4 + k] + hi
            return tuple(accs)

        z = jnp.zeros((16,), jnp.float32)
        accs = lax.fori_loop(0, _L // 8, acc_body, (z,) * 8)
        scale = jnp.float32(1.0 / _L)
        acc_v[b, 0:16] = ((accs[0] + accs[1]) + (accs[2] + accs[3])) * scale
        acc_v[b, 16:32] = ((accs[4] + accs[5]) + (accs[6] + accs[7])) * scale

    fire(0, 0)
    fire(1, 1)

    def pair_body(p, _):
        b0 = 2 * p
        wait_slot(0)
        accumulate(b0, 0)

        @pl.when(p < _CPW // 2 - 1)
        def _():
            fire(b0 + 2, 0)

        wait_slot(1)
        accumulate(b0 + 1, 1)

        @pl.when(p < _CPW // 2 - 1)
        def _():
            fire(b0 + 3, 1)

        return 0

    lax.fori_loop(0, _CPW // 2, pair_body, 0)
    pltpu.sync_copy(acc_v, out_hbm.at[pl.ds(base, _CPW)])


def _head_body(pooled_ref, bvec_ref, probs_ref, risk_ref):
    x = pooled_ref[...]
    logits = x + bvec_ref[...]
    m = jnp.max(logits, axis=-1, keepdims=True)
    e = jnp.exp(logits - m)
    s = jnp.sum(e, axis=-1, keepdims=True)
    probs_ref[...] = (e / s)[:, :_NCL]
    risk_ref[...] = 1.0 / (1.0 + jnp.exp(-x[:, 30:31]))


def _head(pooled, bvec):
    return pl.pallas_call(
        _head_body,
        out_shape=(
            jax.ShapeDtypeStruct((_B, _NCL), jnp.float32),
            jax.ShapeDtypeStruct((_B, 1), jnp.float32),
        ),
    )(pooled, bvec)


def kernel(contract_tokens, emb_table, W, b):
    tokens = contract_tokens.astype(jnp.int32).reshape(_B, 2, _HALF)
    e0 = jnp.zeros((_D, 2), jnp.float32).at[0, 0].set(1.0)
    wp = jnp.concatenate([W, e0], axis=1)
    wp = jnp.stack([wp[:, :16], wp[:, 16:]], axis=2).reshape(_D, _P)
    bvec = jnp.concatenate(
        [b, jnp.full((2,), -1e30, jnp.float32)]).reshape(1, _P)
    proj = _project(emb_table, wp)
    pooled = _pool_kernel(tokens, proj)
    clause_types, risk_score = _head(pooled, bvec)
    return (clause_types, risk_score)


# speedup vs baseline: 1.0952x; 1.0952x over previous
import functools

import jax
import jax.numpy as jnp
from jax import lax
from jax.experimental import pallas as pl
from jax.experimental.pallas import tpu as pltpu
from jax.experimental.pallas import tpu_sc as plsc

_VOCAB = 100000
_D = 512
_NCL = 30
_B = 4096
_L = 200
_P = 32
_C0 = 104
_C1 = _L - _C0

_info = plsc.get_sparse_core_info()
_NC, _NS = _info.num_cores, _info.num_subcores
_NW = _NC * _NS
_CPW = _B // _NW


def _proj_body(emb_ref, wp_ref, out_ref):
    out_ref[...] = jnp.dot(emb_ref[...], wp_ref[...],
                           preferred_element_type=jnp.float32)


def _project(emb_table, wp):
    rows = 2000
    return pl.pallas_call(
        _proj_body,
        grid=(_VOCAB // rows,),
        in_specs=[
            pl.BlockSpec((rows, _D), lambda i: (i, 0)),
            pl.BlockSpec((_D, _P), lambda i: (0, 0)),
        ],
        out_specs=pl.BlockSpec((rows, _P), lambda i: (i, 0)),
        out_shape=jax.ShapeDtypeStruct((_VOCAB, _P), jnp.float32),
    )(emb_table, wp)


_mesh = plsc.VectorSubcoreMesh(core_axis_name="c", subcore_axis_name="s")


@functools.partial(
    pl.kernel,
    out_type=jax.ShapeDtypeStruct((_B, _P), jnp.float32),
    mesh=_mesh,
    scratch_types=[
        pltpu.VMEM((_CPW, _L), jnp.int32),
        pltpu.VMEM((2, _L, _P), jnp.float32),
        pltpu.VMEM((_CPW, _P), jnp.float32),
        pltpu.SemaphoreType.DMA,
        pltpu.SemaphoreType.DMA,
    ],
    compiler_params=pltpu.CompilerParams(use_tc_tiling_on_sc=False),
)
def _pool_kernel(tok_hbm, proj_hbm, out_hbm, idx_v, rows_v, acc_v, sem0, sem1):
    wid = lax.axis_index("s") * _NC + lax.axis_index("c")
    base = wid * _CPW
    pltpu.sync_copy(tok_hbm.at[pl.ds(base, _CPW)], idx_v)
    sems = (sem0, sem1)

    def fire(b, slot):
        pltpu.async_copy(proj_hbm.at[idx_v.at[b, pl.ds(0, _C0)]],
                         rows_v.at[slot, pl.ds(0, _C0)], sems[slot])
        pltpu.async_copy(proj_hbm.at[idx_v.at[b, pl.ds(_C0, _C1)]],
                         rows_v.at[slot, pl.ds(_C0, _C1)], sems[slot])

    def wait_slot(slot):
        pltpu.make_async_copy(proj_hbm.at[pl.ds(0, _L)],
                              rows_v.at[slot], sems[slot]).wait()

    def accumulate(b, slot):
        def acc_body(i, carry):
            accs = list(carry)
            for j in range(8):
                r = i * 8 + j
                k = j % 4
                accs[k] = accs[k] + rows_v[slot, r, 0:16]
                accs[4 + k] = accs[4 + k] + rows_v[slot, r, 16:32]
            return tuple(accs)

        z = jnp.zeros((16,), jnp.float32)
        accs = lax.fori_loop(0, _L // 8, acc_body, (z,) * 8)
        scale = jnp.float32(1.0 / _L)
        acc_v[b, 0:16] = ((accs[0] + accs[1]) + (accs[2] + accs[3])) * scale
        acc_v[b, 16:32] = ((accs[4] + accs[5]) + (accs[6] + accs[7])) * scale

    fire(0, 0)
    fire(1, 1)

    def pair_body(p, _):
        b0 = 2 * p
        wait_slot(0)
        accumulate(b0, 0)

        @pl.when(p < _CPW // 2 - 1)
        def _():
            fire(b0 + 2, 0)

        wait_slot(1)
        accumulate(b0 + 1, 1)

        @pl.when(p < _CPW // 2 - 1)
        def _():
            fire(b0 + 3, 1)

        return 0

    lax.fori_loop(0, _CPW // 2, pair_body, 0)
    pltpu.sync_copy(acc_v, out_hbm.at[pl.ds(base, _CPW)])


def _head_body(pooled_ref, bvec_ref, probs_ref, risk_ref):
    x = pooled_ref[...]
    logits = x + bvec_ref[...]
    m = jnp.max(logits, axis=-1, keepdims=True)
    e = jnp.exp(logits - m)
    s = jnp.sum(e, axis=-1, keepdims=True)
    probs_ref[...] = (e / s)[:, :_NCL]
    risk_ref[...] = 1.0 / (1.0 + jnp.exp(-x[:, 30:31]))


def _head(pooled, bvec):
    return pl.pallas_call(
        _head_body,
        out_shape=(
            jax.ShapeDtypeStruct((_B, _NCL), jnp.float32),
            jax.ShapeDtypeStruct((_B, 1), jnp.float32),
        ),
    )(pooled, bvec)


def kernel(contract_tokens, emb_table, W, b):
    tokens = contract_tokens.astype(jnp.int32)
    e0 = jnp.zeros((_D, 2), jnp.float32).at[0, 0].set(1.0)
    wp = jnp.concatenate([W, e0], axis=1)
    bvec = jnp.concatenate(
        [b, jnp.full((2,), -1e30, jnp.float32)]).reshape(1, _P)
    proj = _project(emb_table, wp)
    pooled = _pool_kernel(tokens, proj)
    clause_types, risk_score = _head(pooled, bvec)
    return (clause_types, risk_score)


# speedup vs baseline: 1.2874x; 1.1755x over previous
import functools

import jax
import jax.numpy as jnp
from jax import lax
from jax.experimental import pallas as pl
from jax.experimental.pallas import tpu as pltpu
from jax.experimental.pallas import tpu_sc as plsc

_VOCAB = 100000
_D = 512
_NCL = 30
_B = 4096
_L = 200
_P = 32
_C0 = 104
_C1 = _L - _C0

_info = plsc.get_sparse_core_info()
_NC, _NS = _info.num_cores, _info.num_subcores
_NW = _NC * _NS
_CPW = _B // _NW


def _proj_body(emb_ref, wp_ref, out_ref):
    x = emb_ref[...].reshape(out_ref.shape[0], 4, _D)
    parts = [jnp.dot(x[:, c, :], wp_ref[...],
                     preferred_element_type=jnp.float32) for c in range(4)]
    out_ref[...] = jnp.concatenate(parts, axis=1)


def _project(emb_table, wp):
    rows = 4000
    return pl.pallas_call(
        _proj_body,
        grid=(_VOCAB // rows,),
        in_specs=[
            pl.BlockSpec((rows, _D), lambda i: (i, 0)),
            pl.BlockSpec((_D, _P), lambda i: (0, 0)),
        ],
        out_specs=pl.BlockSpec((rows // 4, 128), lambda i: (i, 0)),
        out_shape=jax.ShapeDtypeStruct((_VOCAB // 4, 128), jnp.float32),
    )(emb_table, wp)


_mesh = plsc.VectorSubcoreMesh(core_axis_name="c", subcore_axis_name="s")


@functools.partial(
    pl.kernel,
    out_type=jax.ShapeDtypeStruct((_B, _P), jnp.float32),
    mesh=_mesh,
    scratch_types=[
        pltpu.VMEM((_CPW, _L), jnp.int32),
        pltpu.VMEM((2, _L, _P), jnp.float32),
        pltpu.VMEM((_CPW, _P), jnp.float32),
        pltpu.SemaphoreType.DMA,
        pltpu.SemaphoreType.DMA,
    ],
    compiler_params=pltpu.CompilerParams(use_tc_tiling_on_sc=False),
)
def _pool_kernel(tok_hbm, proj_hbm, out_hbm, idx_v, rows_v, acc_v, sem0, sem1):
    wid = lax.axis_index("s") * _NC + lax.axis_index("c")
    base = wid * _CPW
    pltpu.sync_copy(tok_hbm.at[pl.ds(base, _CPW)], idx_v)
    sems = (sem0, sem1)

    def fire(b, slot):
        pltpu.async_copy(proj_hbm.at[idx_v.at[b, pl.ds(0, _C0)]],
                         rows_v.at[slot, pl.ds(0, _C0)], sems[slot])
        pltpu.async_copy(proj_hbm.at[idx_v.at[b, pl.ds(_C0, _C1)]],
                         rows_v.at[slot, pl.ds(_C0, _C1)], sems[slot])

    def wait_slot(slot):
        pltpu.make_async_copy(proj_hbm.at[pl.ds(0, _L)],
                              rows_v.at[slot], sems[slot]).wait()

    def accumulate(b, slot):
        def acc_body(i, carry):
            accs = list(carry)
            for j in range(8):
                r = i * 8 + j
                k = j % 4
                accs[k] = accs[k] + rows_v[slot, r, 0:16]
                accs[4 + k] = accs[4 + k] + rows_v[slot, r, 16:32]
            return tuple(accs)

        z = jnp.zeros((16,), jnp.float32)
        accs = lax.fori_loop(0, _L // 8, acc_body, (z,) * 8)
        scale = jnp.float32(1.0 / _L)
        acc_v[b, 0:16] = ((accs[0] + accs[1]) + (accs[2] + accs[3])) * scale
        acc_v[b, 16:32] = ((accs[4] + accs[5]) + (accs[6] + accs[7])) * scale

    fire(0, 0)
    fire(1, 1)

    def pair_body(p, _):
        b0 = 2 * p
        wait_slot(0)
        accumulate(b0, 0)

        @pl.when(p < _CPW // 2 - 1)
        def _():
            fire(b0 + 2, 0)

        wait_slot(1)
        accumulate(b0 + 1, 1)

        @pl.when(p < _CPW // 2 - 1)
        def _():
            fire(b0 + 3, 1)

        return 0

    lax.fori_loop(0, _CPW // 2, pair_body, 0)
    pltpu.sync_copy(acc_v, out_hbm.at[pl.ds(base, _CPW)])


def _head_body(pooled_ref, bvec_ref, probs_ref, risk_ref):
    x = pooled_ref[...]
    logits = x + bvec_ref[...]
    m = jnp.max(logits, axis=-1, keepdims=True)
    e = jnp.exp(logits - m)
    s = jnp.sum(e, axis=-1, keepdims=True)
    probs_ref[...] = (e / s)[:, :_NCL]
    risk_ref[...] = 1.0 / (1.0 + jnp.exp(-x[:, 30:31]))


def _head(pooled, bvec):
    return pl.pallas_call(
        _head_body,
        out_shape=(
            jax.ShapeDtypeStruct((_B, _NCL), jnp.float32),
            jax.ShapeDtypeStruct((_B, 1), jnp.float32),
        ),
    )(pooled, bvec)


def kernel(contract_tokens, emb_table, W, b):
    tokens = contract_tokens.astype(jnp.int32)
    e0 = jnp.zeros((_D, 2), jnp.float32).at[0, 0].set(1.0)
    wp = jnp.concatenate([W, e0], axis=1)
    bvec = jnp.concatenate(
        [b, jnp.full((2,), -1e30, jnp.float32)]).reshape(1, _P)
    proj = _project(emb_table, wp).reshape(_VOCAB, _P)
    pooled = _pool_kernel(tokens, proj)
    clause_types, risk_score = _head(pooled, bvec)
    return (clause_types, risk_score)


# speedup vs baseline: 1.5243x; 1.1840x over previous
import functools

import jax
import jax.numpy as jnp
from jax import lax
from jax.experimental import pallas as pl
from jax.experimental.pallas import tpu as pltpu
from jax.experimental.pallas import tpu_sc as plsc

_VOCAB = 100000
_D = 512
_NCL = 30
_B = 4096
_L = 200
_P = 32
_C0 = 104
_C1 = _L - _C0

_info = plsc.get_sparse_core_info()
_NC, _NS = _info.num_cores, _info.num_subcores
_NW = _NC * _NS
_CPW = _B // _NW


def _proj_body(emb_ref, wp_ref, out_ref):
    x = emb_ref[...].reshape(out_ref.shape[0], 4, _D)
    parts = [jnp.dot(x[:, c, :], wp_ref[...],
                     preferred_element_type=jnp.float32) for c in range(4)]
    out_ref[...] = jnp.concatenate(parts, axis=1)


def _project(emb_table, wp):
    rows = 4000
    return pl.pallas_call(
        _proj_body,
        grid=(_VOCAB // rows,),
        in_specs=[
            pl.BlockSpec((rows, _D), lambda i: (i, 0)),
            pl.BlockSpec((_D, _P), lambda i: (0, 0)),
        ],
        out_specs=pl.BlockSpec((rows // 4, 128), lambda i: (i, 0)),
        out_shape=jax.ShapeDtypeStruct((_VOCAB // 4, 128), jnp.float32),
    )(emb_table, wp)


_mesh = plsc.VectorSubcoreMesh(core_axis_name="c", subcore_axis_name="s")


@functools.partial(
    pl.kernel,
    out_type=jax.ShapeDtypeStruct((_B, _P), jnp.float32),
    mesh=_mesh,
    scratch_types=[
        pltpu.VMEM((_CPW, _L), jnp.int32),
        pltpu.VMEM((4, _L, _P), jnp.float32),
        pltpu.VMEM((_CPW, _P), jnp.float32),
        pltpu.SemaphoreType.DMA,
        pltpu.SemaphoreType.DMA,
        pltpu.SemaphoreType.DMA,
        pltpu.SemaphoreType.DMA,
    ],
    compiler_params=pltpu.CompilerParams(use_tc_tiling_on_sc=False),
)
def _pool_kernel(tok_hbm, proj_hbm, out_hbm, idx_v, rows_v, acc_v,
                 sem0, sem1, sem2, sem3):
    wid = lax.axis_index("s") * _NC + lax.axis_index("c")
    base = wid * _CPW
    pltpu.sync_copy(tok_hbm.at[pl.ds(base, _CPW)], idx_v)
    sems = (sem0, sem1, sem2, sem3)

    def fire(b, slot):
        pltpu.async_copy(proj_hbm.at[idx_v.at[b, pl.ds(0, _C0)]],
                         rows_v.at[slot, pl.ds(0, _C0)], sems[slot])
        pltpu.async_copy(proj_hbm.at[idx_v.at[b, pl.ds(_C0, _C1)]],
                         rows_v.at[slot, pl.ds(_C0, _C1)], sems[slot])

    def wait_slot(slot):
        pltpu.make_async_copy(proj_hbm.at[pl.ds(0, _L)],
                              rows_v.at[slot], sems[slot]).wait()

    def accumulate(b, slot):
        def acc_body(i, carry):
            accs = list(carry)
            for j in range(8):
                r = i * 8 + j
                k = j % 4
                accs[k] = accs[k] + rows_v[slot, r, 0:16]
                accs[4 + k] = accs[4 + k] + rows_v[slot, r, 16:32]
            return tuple(accs)

        z = jnp.zeros((16,), jnp.float32)
        accs = lax.fori_loop(0, _L // 8, acc_body, (z,) * 8)
        scale = jnp.float32(1.0 / _L)
        acc_v[b, 0:16] = ((accs[0] + accs[1]) + (accs[2] + accs[3])) * scale
        acc_v[b, 16:32] = ((accs[4] + accs[5]) + (accs[6] + accs[7])) * scale

    for s in range(4):
        fire(s, s)

    def quad_body(q, _):
        b0 = 4 * q
        for s in range(4):
            wait_slot(s)
            accumulate(b0 + s, s)

            @pl.when(q < _CPW // 4 - 1)
            def _(s=s):
                fire(b0 + 4 + s, s)

        return 0

    lax.fori_loop(0, _CPW // 4, quad_body, 0)
    pltpu.sync_copy(acc_v, out_hbm.at[pl.ds(base, _CPW)])


def _head_body(pooled_ref, bvec_ref, probs_ref, risk_ref):
    x = pooled_ref[...]
    logits = x + bvec_ref[...]
    m = jnp.max(logits, axis=-1, keepdims=True)
    e = jnp.exp(logits - m)
    s = jnp.sum(e, axis=-1, keepdims=True)
    probs_ref[...] = (e / s)[:, :_NCL]
    risk_ref[...] = 1.0 / (1.0 + jnp.exp(-x[:, 30:31]))


def _head(pooled, bvec):
    return pl.pallas_call(
        _head_body,
        out_shape=(
            jax.ShapeDtypeStruct((_B, _NCL), jnp.float32),
            jax.ShapeDtypeStruct((_B, 1), jnp.float32),
        ),
    )(pooled, bvec)


def kernel(contract_tokens, emb_table, W, b):
    tokens = contract_tokens.astype(jnp.int32)
    e0 = jnp.zeros((_D, 2), jnp.float32).at[0, 0].set(1.0)
    wp = jnp.concatenate([W, e0], axis=1)
    bvec = jnp.concatenate(
        [b, jnp.full((2,), -1e30, jnp.float32)]).reshape(1, _P)
    proj = _project(emb_table, wp).reshape(_VOCAB, _P)
    pooled = _pool_kernel(tokens, proj)
    clause_types, risk_score = _head(pooled, bvec)
    return (clause_types, risk_score)


# speedup vs baseline: 1.5903x; 1.0433x over previous
import functools

import jax
import jax.numpy as jnp
from jax import lax
from jax.experimental import pallas as pl
from jax.experimental.pallas import tpu as pltpu
from jax.experimental.pallas import tpu_sc as plsc

_VOCAB = 100000
_D = 512
_NCL = 30
_B = 4096
_L = 200
_P = 32
_C0 = 104
_C1 = _L - _C0
_NBUF = 8

_info = plsc.get_sparse_core_info()
_NC, _NS = _info.num_cores, _info.num_subcores
_NW = _NC * _NS
_CPW = _B // _NW


def _proj_body(emb_ref, wp_ref, out_ref):
    x = emb_ref[...].reshape(out_ref.shape[0], 4, _D)
    parts = [jnp.dot(x[:, c, :], wp_ref[...],
                     preferred_element_type=jnp.float32) for c in range(4)]
    out_ref[...] = jnp.concatenate(parts, axis=1)


def _project(emb_table, wp):
    rows = 4000
    return pl.pallas_call(
        _proj_body,
        grid=(_VOCAB // rows,),
        in_specs=[
            pl.BlockSpec((rows, _D), lambda i: (i, 0)),
            pl.BlockSpec((_D, _P), lambda i: (0, 0)),
        ],
        out_specs=pl.BlockSpec((rows // 4, 128), lambda i: (i, 0)),
        out_shape=jax.ShapeDtypeStruct((_VOCAB // 4, 128), jnp.float32),
    )(emb_table, wp)


_mesh = plsc.VectorSubcoreMesh(core_axis_name="c", subcore_axis_name="s")


@functools.partial(
    pl.kernel,
    out_type=(jax.ShapeDtypeStruct((_B, _P), jnp.float32),
              jax.ShapeDtypeStruct((_B, 16), jnp.float32)),
    mesh=_mesh,
    scratch_types=[
        pltpu.VMEM((_CPW, _L), jnp.int32),
        pltpu.VMEM((_NBUF, _L, _P), jnp.float32),
        pltpu.VMEM((_CPW, _P), jnp.float32),
        pltpu.VMEM((_CPW, 16), jnp.float32),
        pltpu.VMEM((_P,), jnp.float32),
        *([pltpu.SemaphoreType.DMA] * _NBUF),
    ],
    compiler_params=pltpu.CompilerParams(use_tc_tiling_on_sc=False,
                                         needs_layout_passes=False),
)
def _pool_kernel(tok_hbm, proj_hbm, bvec_hbm, probs_hbm, risk_hbm,
                 idx_v, rows_v, probs_v, risk_v, bvec_v, *sems):
    wid = lax.axis_index("s") * _NC + lax.axis_index("c")
    base = wid * _CPW
    pltpu.sync_copy(tok_hbm.at[pl.ds(base, _CPW)], idx_v)
    pltpu.sync_copy(bvec_hbm, bvec_v)
    bv_lo = bvec_v[0:16]
    bv_hi = bvec_v[16:32]

    def fire(b, slot):
        pltpu.async_copy(proj_hbm.at[idx_v.at[b, pl.ds(0, _C0)]],
                         rows_v.at[slot, pl.ds(0, _C0)], sems[slot])
        pltpu.async_copy(proj_hbm.at[idx_v.at[b, pl.ds(_C0, _C1)]],
                         rows_v.at[slot, pl.ds(_C0, _C1)], sems[slot])

    def wait_slot(slot):
        pltpu.make_async_copy(proj_hbm.at[pl.ds(0, _L)],
                              rows_v.at[slot], sems[slot]).wait()

    def accumulate(b, slot):
        def acc_body(i, carry):
            accs = list(carry)
            for j in range(8):
                r = i * 8 + j
                k = j % 4
                accs[k] = accs[k] + rows_v[slot, r, 0:16]
                accs[4 + k] = accs[4 + k] + rows_v[slot, r, 16:32]
            return tuple(accs)

        z = jnp.zeros((16,), jnp.float32)
        accs = lax.fori_loop(0, _L // 8, acc_body, (z,) * 8)
        scale = jnp.float32(1.0 / _L)
        a_lo = ((accs[0] + accs[1]) + (accs[2] + accs[3])) * scale
        a_hi = ((accs[4] + accs[5]) + (accs[6] + accs[7])) * scale
        l_lo = a_lo + bv_lo
        l_hi = a_hi + bv_hi
        m = jnp.maximum(jnp.max(l_lo), jnp.max(l_hi))
        e_lo = jnp.exp(l_lo - m)
        e_hi = jnp.exp(l_hi - m)
        s = jnp.sum(e_lo) + jnp.sum(e_hi)
        probs_v[b, 0:16] = e_lo / s
        probs_v[b, 16:32] = e_hi / s
        risk_v[b, 0:16] = 1.0 / (1.0 + jnp.exp(-a_hi))

    for s in range(_NBUF):
        fire(s, s)

    def ring_body(q, _):
        b0 = _NBUF * q
        for s in range(_NBUF):
            wait_slot(s)
            accumulate(b0 + s, s)

            @pl.when(q < _CPW // _NBUF - 1)
            def _(s=s):
                fire(b0 + _NBUF + s, s)

        return 0

    lax.fori_loop(0, _CPW // _NBUF, ring_body, 0)
    pltpu.sync_copy(probs_v, probs_hbm.at[pl.ds(base, _CPW)])
    pltpu.sync_copy(risk_v, risk_hbm.at[pl.ds(base, _CPW)])


def kernel(contract_tokens, emb_table, W, b):
    tokens = contract_tokens.astype(jnp.int32)
    e0 = jnp.zeros((_D, 2), jnp.float32).at[0, 0].set(1.0)
    wp = jnp.concatenate([W, e0], axis=1)
    bvec = jnp.concatenate([b, jnp.full((2,), -1e30, jnp.float32)])
    proj = _project(emb_table, wp).reshape(_VOCAB, _P)
    probs32, risk16 = _pool_kernel(tokens, proj, bvec)
    return (probs32[:, :_NCL], risk16[:, 14:15])


# speedup vs baseline: 1.5905x; 1.0001x over previous
import functools

import jax
import jax.numpy as jnp
from jax import lax
from jax.experimental import pallas as pl
from jax.experimental.pallas import tpu as pltpu
from jax.experimental.pallas import tpu_sc as plsc

_VOCAB = 100000
_D = 512
_NCL = 30
_B = 4096
_L = 200
_P = 32
_C0 = 104
_C1 = _L - _C0
_NBUF = 8

_info = plsc.get_sparse_core_info()
_NC, _NS = _info.num_cores, _info.num_subcores
_NW = _NC * _NS
_CPW = _B // _NW


def _proj_body(emb_ref, wp_ref, out_ref):
    x = emb_ref[...].reshape(out_ref.shape[0], 4, _D)
    for c in range(4):
        out_ref[:, 32 * c:32 * (c + 1)] = jnp.dot(
            x[:, c, :], wp_ref[...], preferred_element_type=jnp.float32)


def _project(emb_table, wp):
    rows = 4000
    return pl.pallas_call(
        _proj_body,
        grid=(_VOCAB // rows,),
        in_specs=[
            pl.BlockSpec((rows, _D), lambda i: (i, 0)),
            pl.BlockSpec((_D, _P), lambda i: (0, 0)),
        ],
        out_specs=pl.BlockSpec((rows // 4, 128), lambda i: (i, 0)),
        out_shape=jax.ShapeDtypeStruct((_VOCAB // 4, 128), jnp.float32),
    )(emb_table, wp)


_mesh = plsc.VectorSubcoreMesh(core_axis_name="c", subcore_axis_name="s")


@functools.partial(
    pl.kernel,
    out_type=(jax.ShapeDtypeStruct((_B, _P), jnp.float32),
              jax.ShapeDtypeStruct((_B, 16), jnp.float32)),
    mesh=_mesh,
    scratch_types=[
        pltpu.VMEM((_CPW, _L), jnp.int32),
        pltpu.VMEM((_NBUF, _L, _P), jnp.float32),
        pltpu.VMEM((_CPW, _P), jnp.float32),
        pltpu.VMEM((_CPW, 16), jnp.float32),
        pltpu.VMEM((_P,), jnp.float32),
        *([pltpu.SemaphoreType.DMA] * _NBUF),
    ],
    compiler_params=pltpu.CompilerParams(use_tc_tiling_on_sc=False,
                                         needs_layout_passes=False),
)
def _pool_kernel(tok_hbm, proj_hbm, bvec_hbm, probs_hbm, risk_hbm,
                 idx_v, rows_v, probs_v, risk_v, bvec_v, *sems):
    wid = lax.axis_index("s") * _NC + lax.axis_index("c")
    base = wid * _CPW
    pltpu.sync_copy(tok_hbm.at[pl.ds(base, _CPW)], idx_v)
    pltpu.sync_copy(bvec_hbm, bvec_v)
    bv_lo = bvec_v[0:16]
    bv_hi = bvec_v[16:32]

    def fire(b, slot):
        pltpu.async_copy(proj_hbm.at[idx_v.at[b, pl.ds(0, _C0)]],
                         rows_v.at[slot, pl.ds(0, _C0)], sems[slot])
        pltpu.async_copy(proj_hbm.at[idx_v.at[b, pl.ds(_C0, _C1)]],
                         rows_v.at[slot, pl.ds(_C0, _C1)], sems[slot])

    def wait_slot(slot):
        pltpu.make_async_copy(proj_hbm.at[pl.ds(0, _L)],
                              rows_v.at[slot], sems[slot]).wait()

    def accumulate(b, slot):
        def acc_body(i, carry):
            accs = list(carry)
            for j in range(8):
                r = i * 8 + j
                k = j % 4
                accs[k] = accs[k] + rows_v[slot, r, 0:16]
                accs[4 + k] = accs[4 + k] + rows_v[slot, r, 16:32]
            return tuple(accs)

        z = jnp.zeros((16,), jnp.float32)
        accs = lax.fori_loop(0, _L // 8, acc_body, (z,) * 8)
        scale = jnp.float32(1.0 / _L)
        a_lo = ((accs[0] + accs[1]) + (accs[2] + accs[3])) * scale
        a_hi = ((accs[4] + accs[5]) + (accs[6] + accs[7])) * scale
        l_lo = a_lo + bv_lo
        l_hi = a_hi + bv_hi
        m = jnp.maximum(jnp.max(l_lo), jnp.max(l_hi))
        e_lo = jnp.exp(l_lo - m)
        e_hi = jnp.exp(l_hi - m)
        s = jnp.sum(e_lo) + jnp.sum(e_hi)
        probs_v[b, 0:16] = e_lo / s
        probs_v[b, 16:32] = e_hi / s
        risk_v[b, 0:16] = 1.0 / (1.0 + jnp.exp(-a_hi))

    for s in range(_NBUF):
        fire(s, s)

    def ring_body(q, _):
        b0 = _NBUF * q
        for s in range(_NBUF):
            wait_slot(s)
            accumulate(b0 + s, s)

            @pl.when(q < _CPW // _NBUF - 1)
            def _(s=s):
                fire(b0 + _NBUF + s, s)

        return 0

    lax.fori_loop(0, _CPW // _NBUF, ring_body, 0)
    pltpu.sync_copy(probs_v, probs_hbm.at[pl.ds(base, _CPW)])
    pltpu.sync_copy(risk_v, risk_hbm.at[pl.ds(base, _CPW)])


def kernel(contract_tokens, emb_table, W, b):
    tokens = contract_tokens.astype(jnp.int32)
    e0 = jnp.zeros((_D, 2), jnp.float32).at[0, 0].set(1.0)
    wp = jnp.concatenate([W, e0], axis=1)
    bvec = jnp.concatenate([b, jnp.full((2,), -1e30, jnp.float32)])
    proj = _project(emb_table, wp).reshape(_VOCAB, _P)
    probs32, risk16 = _pool_kernel(tokens, proj, bvec)
    return (probs32[:, :_NCL], risk16[:, 14:15])
